# baseline (speedup 1.0000x reference)
import functools

import jax
import jax.numpy as jnp
from jax import lax
from jax.experimental import pallas as pl
from jax.experimental.pallas import tpu as pltpu
from jax.experimental.pallas import tpu_sc as plsc

N = 10000
E = 320000
D_FEAT = 128
D_HID = 128
H = 256
N_OBJ = 32
N_CLS = 8

ROW_BLK = 1000

SC_CORES = 2
SC_SUBCORES = 16
NW = SC_CORES * SC_SUBCORES
PER_W = E // NW
CHUNK = 400
GROUPS = CHUNK // 16
NCH = PER_W // CHUNK


def _mean_body(hid_ref, out_ref):
    i = pl.program_id(0)

    @pl.when(i == 0)
    def _init():
        out_ref[...] = jnp.zeros_like(out_ref)

    out_ref[...] += jnp.sum(hid_ref[...], axis=0, keepdims=True)

    @pl.when(i == pl.num_programs(0) - 1)
    def _fin():
        out_ref[...] = out_ref[...] * (1.0 / N)


def _col_mean(node_hidden):
    return pl.pallas_call(
        _mean_body,
        grid=(N // ROW_BLK,),
        in_specs=[pl.BlockSpec((ROW_BLK, D_HID), lambda i: (i, 0))],
        out_specs=pl.BlockSpec((1, D_HID), lambda i: (0, 0)),
        out_shape=jax.ShapeDtypeStruct((1, D_HID), jnp.float32),
    )(node_hidden)


def _bdot(a, b):
    return jnp.dot(a.astype(jnp.bfloat16), b.astype(jnp.bfloat16),
                   preferred_element_type=jnp.float32)


def _mlp_body(last_relu, feat_ref, hid_ref, mean_ref,
              W1, b1, W2, b2, W3, b3, out_ref):
    x = jnp.concatenate(
        [feat_ref[...], hid_ref[...],
         jnp.broadcast_to(mean_ref[...], (ROW_BLK, D_HID))], axis=1)
    h = _bdot(x, W1[...]) + b1[...]
    h = jnp.maximum(_bdot(h, W2[...]) + b2[...], 0.0)
    o = _bdot(h, W3[...]) + b3[...]
    out_ref[...] = jnp.maximum(o, 0.0) if last_relu else o


def _full(shape):
    return pl.BlockSpec(shape, lambda i: tuple(0 for _ in shape))


def _mlp(last_relu, d_out, node_features, node_hidden, mean_h,
         W1, b1, W2, b2, W3, b3):
    return pl.pallas_call(
        functools.partial(_mlp_body, last_relu),
        grid=(N // ROW_BLK,),
        in_specs=[
            pl.BlockSpec((ROW_BLK, D_FEAT), lambda i: (i, 0)),
            pl.BlockSpec((ROW_BLK, D_HID), lambda i: (i, 0)),
            _full((1, D_HID)),
            _full((D_FEAT + 2 * D_HID, H)), _full((1, H)),
            _full((H, H)), _full((1, H)),
            _full((H, d_out)), _full((1, d_out)),
        ],
        out_specs=pl.BlockSpec((ROW_BLK, d_out), lambda i: (i, 0)),
        out_shape=jax.ShapeDtypeStruct((N, d_out), jnp.float32),
    )(node_features, node_hidden, mean_h, W1, b1, W2, b2, W3, b3)


def _edge_body(table_hbm, src_hbm, dst_hbm, out_hbm,
               idx_s, idx_d, rs_a, rd_a, rs_b, rd_b, out_v,
               ss_a, sd_a, ss_b, sd_b):
    wid = lax.axis_index("s") * SC_CORES + lax.axis_index("c")
    base = wid * PER_W
    pltpu.sync_copy(src_hbm.at[pl.ds(base, PER_W)], idx_s)
    pltpu.sync_copy(dst_hbm.at[pl.ds(base, PER_W)], idx_d)
    lane = lax.iota(jnp.int32, 16)

    def start(c, rs, rd, ss, sd):
        off = c * CHUNK
        pltpu.make_async_copy(
            table_hbm.at[idx_s.at[pl.ds(off, CHUNK)]], rs, ss).start()
        pltpu.make_async_copy(
            table_hbm.at[idx_d.at[pl.ds(off, CHUNK)]], rd, sd).start()

    def wait(rs, rd, ss, sd):
        pltpu.make_async_copy(
            table_hbm.at[idx_s.at[pl.ds(0, CHUNK)]], rs, ss).wait()
        pltpu.make_async_copy(
            table_hbm.at[idx_d.at[pl.ds(0, CHUNK)]], rd, sd).wait()

    def compute(c, rs, rd):
        def group_body(g, carry):
            row_ids = g * 16 + lane
            acc = jnp.zeros((16,), jnp.float32)
            for k in range(N_OBJ):
                col = jnp.bitwise_and(lane + k, N_OBJ - 1)
                acc = acc + (plsc.load_gather(rs, [row_ids, col]) *
                             plsc.load_gather(rd, [row_ids, col]))
            out_v[pl.ds(c * CHUNK + g * 16, 16)] = acc
            return carry
        lax.fori_loop(0, GROUPS, group_body, 0)

    start(0, rs_a, rd_a, ss_a, sd_a)

    def pair_body(i, carry):
        c0 = 2 * i

        @pl.when(c0 + 1 < NCH)
        def _():
            start(c0 + 1, rs_b, rd_b, ss_b, sd_b)

        wait(rs_a, rd_a, ss_a, sd_a)
        compute(c0, rs_a, rd_a)

        @pl.when(c0 + 2 < NCH)
        def _():
            start(c0 + 2, rs_a, rd_a, ss_a, sd_a)

        @pl.when(c0 + 1 < NCH)
        def _():
            wait(rs_b, rd_b, ss_b, sd_b)
            compute(c0 + 1, rs_b, rd_b)

        return carry

    lax.fori_loop(0, (NCH + 1) // 2, pair_body, 0)
    pltpu.sync_copy(out_v, out_hbm.at[pl.ds(base, PER_W)])


@functools.cache
def _edge_scores():
    return pl.kernel(
        _edge_body,
        out_type=jax.ShapeDtypeStruct((E,), jnp.float32),
        mesh=plsc.VectorSubcoreMesh(
            core_axis_name="c", subcore_axis_name="s",
            num_cores=SC_CORES, num_subcores=SC_SUBCORES),
        compiler_params=pltpu.CompilerParams(
            needs_layout_passes=False, use_tc_tiling_on_sc=False),
        scratch_types=[
            pltpu.VMEM((PER_W,), jnp.int32),
            pltpu.VMEM((PER_W,), jnp.int32),
            pltpu.VMEM((CHUNK, N_OBJ), jnp.float32),
            pltpu.VMEM((CHUNK, N_OBJ), jnp.float32),
            pltpu.VMEM((CHUNK, N_OBJ), jnp.float32),
            pltpu.VMEM((CHUNK, N_OBJ), jnp.float32),
            pltpu.VMEM((PER_W,), jnp.float32),
            pltpu.SemaphoreType.DMA,
            pltpu.SemaphoreType.DMA,
            pltpu.SemaphoreType.DMA,
            pltpu.SemaphoreType.DMA,
        ],
    )


def kernel(node_features, node_hidden, edge_index,
           obj_W1, obj_b1, obj_W2, obj_b2, obj_W3, obj_b3,
           nc_W1, nc_b1, nc_W2, nc_b2, nc_W3, nc_b3):
    mean_h = _col_mean(node_hidden)
    obj_pred = _mlp(True, N_OBJ, node_features, node_hidden, mean_h,
                    obj_W1, obj_b1.reshape(1, H), obj_W2, obj_b2.reshape(1, H),
                    obj_W3, obj_b3.reshape(1, N_OBJ))
    src = edge_index[0].astype(jnp.int32)
    dst = edge_index[1].astype(jnp.int32)
    edge_pred = _edge_scores()(obj_pred, src, dst)
    node_pred = _mlp(False, N_CLS, node_features, node_hidden, mean_h,
                     nc_W1, nc_b1.reshape(1, H), nc_W2, nc_b2.reshape(1, H),
                     nc_W3, nc_b3.reshape(1, N_CLS))
    return obj_pred, edge_pred, node_pred


# speedup vs baseline: 1.1653x; 1.1653x over previous
import functools

import jax
import jax.numpy as jnp
from jax import lax
from jax.experimental import pallas as pl
from jax.experimental.pallas import tpu as pltpu
from jax.experimental.pallas import tpu_sc as plsc

N = 10000
E = 320000
D_FEAT = 128
D_HID = 128
H = 256
N_OBJ = 32
N_CLS = 8

ROW_BLK = 2000

SC_CORES = 2
SC_SUBCORES = 16
NW = SC_CORES * SC_SUBCORES
PER_W = E // NW
CHUNK = 400
GROUPS = CHUNK // 16
NCH = PER_W // CHUNK


def _mean_body(hid_ref, out_ref):
    i = pl.program_id(0)

    @pl.when(i == 0)
    def _init():
        out_ref[...] = jnp.zeros_like(out_ref)

    out_ref[...] += jnp.sum(hid_ref[...], axis=0, keepdims=True)

    @pl.when(i == pl.num_programs(0) - 1)
    def _fin():
        out_ref[...] = out_ref[...] * (1.0 / N)


def _col_mean(node_hidden):
    return pl.pallas_call(
        _mean_body,
        grid=(N // ROW_BLK,),
        in_specs=[pl.BlockSpec((ROW_BLK, D_HID), lambda i: (i, 0))],
        out_specs=pl.BlockSpec((1, D_HID), lambda i: (0, 0)),
        out_shape=jax.ShapeDtypeStruct((1, D_HID), jnp.float32),
    )(node_hidden)


def _bdot(a, b):
    return jnp.dot(a.astype(jnp.bfloat16), b.astype(jnp.bfloat16),
                   preferred_element_type=jnp.float32)


def _mlp_body(last_relu, feat_ref, hid_ref, mean_ref,
              W1, b1, W2, b2, W3, b3, out_ref):
    x = jnp.concatenate(
        [feat_ref[...], hid_ref[...],
         jnp.broadcast_to(mean_ref[...], (ROW_BLK, D_HID))], axis=1)
    h = _bdot(x, W1[...]) + b1[...]
    h = jnp.maximum(_bdot(h, W2[...]) + b2[...], 0.0)
    o = _bdot(h, W3[...]) + b3[...]
    out_ref[...] = jnp.maximum(o, 0.0) if last_relu else o


def _full(shape):
    return pl.BlockSpec(shape, lambda i: tuple(0 for _ in shape))


def _mlp(last_relu, d_out, node_features, node_hidden, mean_h,
         W1, b1, W2, b2, W3, b3):
    return pl.pallas_call(
        functools.partial(_mlp_body, last_relu),
        grid=(N // ROW_BLK,),
        in_specs=[
            pl.BlockSpec((ROW_BLK, D_FEAT), lambda i: (i, 0)),
            pl.BlockSpec((ROW_BLK, D_HID), lambda i: (i, 0)),
            _full((1, D_HID)),
            _full((D_FEAT + 2 * D_HID, H)), _full((1, H)),
            _full((H, H)), _full((1, H)),
            _full((H, d_out)), _full((1, d_out)),
        ],
        out_specs=pl.BlockSpec((ROW_BLK, d_out), lambda i: (i, 0)),
        out_shape=jax.ShapeDtypeStruct((N, d_out), jnp.float32),
    )(node_features, node_hidden, mean_h, W1, b1, W2, b2, W3, b3)


def _edge_body(table_hbm, ei_hbm, out_hbm,
               idx_s, idx_d, rs_a, rd_a, rs_b, rd_b, out_v,
               ss_a, sd_a, ss_b, sd_b):
    wid = lax.axis_index("s") * SC_CORES + lax.axis_index("c")
    base = wid * PER_W
    pltpu.sync_copy(ei_hbm.at[0, pl.ds(base, PER_W)], idx_s)
    pltpu.sync_copy(ei_hbm.at[1, pl.ds(base, PER_W)], idx_d)
    lane = lax.iota(jnp.int32, 16)

    def start(c, rs, rd, ss, sd):
        off = c * CHUNK
        pltpu.make_async_copy(
            table_hbm.at[idx_s.at[pl.ds(off, CHUNK)]], rs, ss).start()
        pltpu.make_async_copy(
            table_hbm.at[idx_d.at[pl.ds(off, CHUNK)]], rd, sd).start()

    def wait(rs, rd, ss, sd):
        pltpu.make_async_copy(
            table_hbm.at[idx_s.at[pl.ds(0, CHUNK)]], rs, ss).wait()
        pltpu.make_async_copy(
            table_hbm.at[idx_d.at[pl.ds(0, CHUNK)]], rd, sd).wait()

    def compute(c, rs, rd):
        def group_body(g, carry):
            row_ids = g * 16 + lane
            accs = [jnp.zeros((16,), jnp.float32) for _ in range(4)]
            for k in range(N_OBJ):
                col = jnp.bitwise_and(lane + k, N_OBJ - 1)
                accs[k % 4] = accs[k % 4] + (
                    plsc.load_gather(rs, [row_ids, col]) *
                    plsc.load_gather(rd, [row_ids, col]))
            out_v[pl.ds(c * CHUNK + g * 16, 16)] = (
                (accs[0] + accs[1]) + (accs[2] + accs[3]))
            return carry
        lax.fori_loop(0, GROUPS, group_body, 0)

    start(0, rs_a, rd_a, ss_a, sd_a)

    def pair_body(i, carry):
        c0 = 2 * i

        @pl.when(c0 + 1 < NCH)
        def _():
            start(c0 + 1, rs_b, rd_b, ss_b, sd_b)

        wait(rs_a, rd_a, ss_a, sd_a)
        compute(c0, rs_a, rd_a)

        @pl.when(c0 + 2 < NCH)
        def _():
            start(c0 + 2, rs_a, rd_a, ss_a, sd_a)

        @pl.when(c0 + 1 < NCH)
        def _():
            wait(rs_b, rd_b, ss_b, sd_b)
            compute(c0 + 1, rs_b, rd_b)

        return carry

    lax.fori_loop(0, (NCH + 1) // 2, pair_body, 0)
    pltpu.sync_copy(out_v, out_hbm.at[pl.ds(base, PER_W)])


@functools.cache
def _edge_scores():
    return pl.kernel(
        _edge_body,
        out_type=jax.ShapeDtypeStruct((E,), jnp.float32),
        mesh=plsc.VectorSubcoreMesh(
            core_axis_name="c", subcore_axis_name="s",
            num_cores=SC_CORES, num_subcores=SC_SUBCORES),
        compiler_params=pltpu.CompilerParams(
            needs_layout_passes=False, use_tc_tiling_on_sc=False),
        scratch_types=[
            pltpu.VMEM((PER_W,), jnp.int32),
            pltpu.VMEM((PER_W,), jnp.int32),
            pltpu.VMEM((CHUNK, N_OBJ), jnp.float32),
            pltpu.VMEM((CHUNK, N_OBJ), jnp.float32),
            pltpu.VMEM((CHUNK, N_OBJ), jnp.float32),
            pltpu.VMEM((CHUNK, N_OBJ), jnp.float32),
            pltpu.VMEM((PER_W,), jnp.float32),
            pltpu.SemaphoreType.DMA,
            pltpu.SemaphoreType.DMA,
            pltpu.SemaphoreType.DMA,
            pltpu.SemaphoreType.DMA,
        ],
    )


def kernel(node_features, node_hidden, edge_index,
           obj_W1, obj_b1, obj_W2, obj_b2, obj_W3, obj_b3,
           nc_W1, nc_b1, nc_W2, nc_b2, nc_W3, nc_b3):
    mean_h = _col_mean(node_hidden)
    obj_pred = _mlp(True, N_OBJ, node_features, node_hidden, mean_h,
                    obj_W1, obj_b1.reshape(1, H), obj_W2, obj_b2.reshape(1, H),
                    obj_W3, obj_b3.reshape(1, N_OBJ))
    edge_pred = _edge_scores()(obj_pred, edge_index.astype(jnp.int32))
    node_pred = _mlp(False, N_CLS, node_features, node_hidden, mean_h,
                     nc_W1, nc_b1.reshape(1, H), nc_W2, nc_b2.reshape(1, H),
                     nc_W3, nc_b3.reshape(1, N_CLS))
    return obj_pred, edge_pred, node_pred


# speedup vs baseline: 1.1963x; 1.0266x over previous
import functools

import jax
import jax.numpy as jnp
from jax import lax
from jax.experimental import pallas as pl
from jax.experimental.pallas import tpu as pltpu
from jax.experimental.pallas import tpu_sc as plsc

N = 10000
E = 320000
D_FEAT = 128
D_HID = 128
H = 256
N_OBJ = 32
N_CLS = 8

ROW_BLK = 2000

SC_CORES = 2
SC_SUBCORES = 16
NW = SC_CORES * SC_SUBCORES
PER_W = E // NW
CHUNK = 400
GROUPS = CHUNK // 16
NCH = PER_W // CHUNK


def _mean_body(hid_ref, out_ref):
    i = pl.program_id(0)

    @pl.when(i == 0)
    def _init():
        out_ref[...] = jnp.zeros_like(out_ref)

    out_ref[...] += jnp.sum(hid_ref[...], axis=0, keepdims=True)

    @pl.when(i == pl.num_programs(0) - 1)
    def _fin():
        out_ref[...] = out_ref[...] * (1.0 / N)


def _col_mean(node_hidden):
    return pl.pallas_call(
        _mean_body,
        grid=(N // ROW_BLK,),
        in_specs=[pl.BlockSpec((ROW_BLK, D_HID), lambda i: (i, 0))],
        out_specs=pl.BlockSpec((1, D_HID), lambda i: (0, 0)),
        out_shape=jax.ShapeDtypeStruct((1, D_HID), jnp.float32),
    )(node_hidden)


def _bdot(a, b):
    return jnp.dot(a.astype(jnp.bfloat16), b.astype(jnp.bfloat16),
                   preferred_element_type=jnp.float32)


def _mlp_body(last_relu, feat_ref, hid_ref, mean_ref,
              W1, b1, W2, b2, W3, b3, out_ref, *maybe_bf16_ref):
    x = jnp.concatenate(
        [feat_ref[...], hid_ref[...],
         jnp.broadcast_to(mean_ref[...], (ROW_BLK, D_HID))], axis=1)
    h = _bdot(x, W1[...]) + b1[...]
    h = jnp.maximum(_bdot(h, W2[...]) + b2[...], 0.0)
    o = _bdot(h, W3[...]) + b3[...]
    o = jnp.maximum(o, 0.0) if last_relu else o
    out_ref[...] = o
    if maybe_bf16_ref:
        maybe_bf16_ref[0][...] = o.astype(jnp.bfloat16)


def _full(shape):
    return pl.BlockSpec(shape, lambda i: tuple(0 for _ in shape))


def _mlp(last_relu, d_out, bf16_copy, node_features, node_hidden, mean_h,
         W1, b1, W2, b2, W3, b3):
    out_specs = [pl.BlockSpec((ROW_BLK, d_out), lambda i: (i, 0))]
    out_shape = [jax.ShapeDtypeStruct((N, d_out), jnp.float32)]
    if bf16_copy:
        out_specs.append(pl.BlockSpec((ROW_BLK, d_out), lambda i: (i, 0)))
        out_shape.append(jax.ShapeDtypeStruct((N, d_out), jnp.bfloat16))
    return pl.pallas_call(
        functools.partial(_mlp_body, last_relu),
        grid=(N // ROW_BLK,),
        in_specs=[
            pl.BlockSpec((ROW_BLK, D_FEAT), lambda i: (i, 0)),
            pl.BlockSpec((ROW_BLK, D_HID), lambda i: (i, 0)),
            _full((1, D_HID)),
            _full((D_FEAT + 2 * D_HID, H)), _full((1, H)),
            _full((H, H)), _full((1, H)),
            _full((H, d_out)), _full((1, d_out)),
        ],
        out_specs=out_specs,
        out_shape=out_shape,
    )(node_features, node_hidden, mean_h, W1, b1, W2, b2, W3, b3)


def _edge_body(table_hbm, ei_hbm, out_hbm,
               idx_s, idx_d, rs_a, rd_a, rs_b, rd_b, out_v,
               ss_a, sd_a, ss_b, sd_b):
    wid = lax.axis_index("s") * SC_CORES + lax.axis_index("c")
    base = wid * PER_W
    pltpu.sync_copy(ei_hbm.at[0, pl.ds(base, PER_W)], idx_s)
    pltpu.sync_copy(ei_hbm.at[1, pl.ds(base, PER_W)], idx_d)
    lane = lax.iota(jnp.int32, 16)

    def start(c, rs, rd, ss, sd):
        off = c * CHUNK
        pltpu.make_async_copy(
            table_hbm.at[idx_s.at[pl.ds(off, CHUNK)]], rs, ss).start()
        pltpu.make_async_copy(
            table_hbm.at[idx_d.at[pl.ds(off, CHUNK)]], rd, sd).start()

    def wait(rs, rd, ss, sd):
        pltpu.make_async_copy(
            table_hbm.at[idx_s.at[pl.ds(0, CHUNK)]], rs, ss).wait()
        pltpu.make_async_copy(
            table_hbm.at[idx_d.at[pl.ds(0, CHUNK)]], rd, sd).wait()

    def compute(c, rs, rd):
        def group_body(g, carry):
            row_ids = g * 16 + lane
            accs = [jnp.zeros((16,), jnp.float32) for _ in range(4)]
            for k in range(N_OBJ // 2):
                col = jnp.bitwise_and(lane + k, N_OBJ // 2 - 1)
                sv = plsc.bitcast(
                    plsc.load_gather(rs, [row_ids, col]), jnp.bfloat16)
                dv = plsc.bitcast(
                    plsc.load_gather(rd, [row_ids, col]), jnp.bfloat16)
                slo, shi = plsc.unpack(
                    sv, format=plsc.PackFormat.INTERLEAVED,
                    preferred_element_type=jnp.float32)
                dlo, dhi = plsc.unpack(
                    dv, format=plsc.PackFormat.INTERLEAVED,
                    preferred_element_type=jnp.float32)
                accs[(2 * k) % 4] = accs[(2 * k) % 4] + slo * dlo
                accs[(2 * k + 1) % 4] = accs[(2 * k + 1) % 4] + shi * dhi
            out_v[pl.ds(c * CHUNK + g * 16, 16)] = (
                (accs[0] + accs[1]) + (accs[2] + accs[3]))
            return carry
        lax.fori_loop(0, GROUPS, group_body, 0)

    start(0, rs_a, rd_a, ss_a, sd_a)

    def pair_body(i, carry):
        c0 = 2 * i

        @pl.when(c0 + 1 < NCH)
        def _():
            start(c0 + 1, rs_b, rd_b, ss_b, sd_b)

        wait(rs_a, rd_a, ss_a, sd_a)
        compute(c0, rs_a, rd_a)

        @pl.when(c0 + 2 < NCH)
        def _():
            start(c0 + 2, rs_a, rd_a, ss_a, sd_a)

        @pl.when(c0 + 1 < NCH)
        def _():
            wait(rs_b, rd_b, ss_b, sd_b)
            compute(c0 + 1, rs_b, rd_b)

        return carry

    lax.fori_loop(0, (NCH + 1) // 2, pair_body, 0)
    pltpu.sync_copy(out_v, out_hbm.at[pl.ds(base, PER_W)])


@functools.cache
def _edge_scores():
    return pl.kernel(
        _edge_body,
        out_type=jax.ShapeDtypeStruct((E,), jnp.float32),
        mesh=plsc.VectorSubcoreMesh(
            core_axis_name="c", subcore_axis_name="s",
            num_cores=SC_CORES, num_subcores=SC_SUBCORES),
        compiler_params=pltpu.CompilerParams(
            needs_layout_passes=False, use_tc_tiling_on_sc=False),
        scratch_types=[
            pltpu.VMEM((PER_W,), jnp.int32),
            pltpu.VMEM((PER_W,), jnp.int32),
            pltpu.VMEM((CHUNK, N_OBJ // 2), jnp.int32),
            pltpu.VMEM((CHUNK, N_OBJ // 2), jnp.int32),
            pltpu.VMEM((CHUNK, N_OBJ // 2), jnp.int32),
            pltpu.VMEM((CHUNK, N_OBJ // 2), jnp.int32),
            pltpu.VMEM((PER_W,), jnp.float32),
            pltpu.SemaphoreType.DMA,
            pltpu.SemaphoreType.DMA,
            pltpu.SemaphoreType.DMA,
            pltpu.SemaphoreType.DMA,
        ],
    )


def kernel(node_features, node_hidden, edge_index,
           obj_W1, obj_b1, obj_W2, obj_b2, obj_W3, obj_b3,
           nc_W1, nc_b1, nc_W2, nc_b2, nc_W3, nc_b3):
    mean_h = _col_mean(node_hidden)
    obj_pred, obj_bf16 = _mlp(
        True, N_OBJ, True, node_features, node_hidden, mean_h,
        obj_W1, obj_b1.reshape(1, H), obj_W2, obj_b2.reshape(1, H),
        obj_W3, obj_b3.reshape(1, N_OBJ))
    table = jax.lax.bitcast_convert_type(
        obj_bf16.reshape(N, N_OBJ // 2, 2), jnp.int32)
    edge_pred = _edge_scores()(table, edge_index)
    [node_pred] = _mlp(
        False, N_CLS, False, node_features, node_hidden, mean_h,
        nc_W1, nc_b1.reshape(1, H), nc_W2, nc_b2.reshape(1, H),
        nc_W3, nc_b3.reshape(1, N_CLS))
    return obj_pred, edge_pred, node_pred


# speedup vs baseline: 1.3632x; 1.1395x over previous
import functools

import jax
import jax.numpy as jnp
from jax import lax
from jax.experimental import pallas as pl
from jax.experimental.pallas import tpu as pltpu
from jax.experimental.pallas import tpu_sc as plsc

N = 10000
E = 320000
D_FEAT = 128
D_HID = 128
H = 256
N_OBJ = 32
N_CLS = 8

ROW_BLK = 2000

SC_CORES = 2
SC_SUBCORES = 16
NW = SC_CORES * SC_SUBCORES
PER_W = E // NW
CHUNK = 400
GROUPS = CHUNK // 16
NCH = PER_W // CHUNK


def _mean_body(hid_ref, out_ref):
    i = pl.program_id(0)

    @pl.when(i == 0)
    def _init():
        out_ref[...] = jnp.zeros_like(out_ref)

    out_ref[...] += jnp.sum(hid_ref[...], axis=0, keepdims=True)

    @pl.when(i == pl.num_programs(0) - 1)
    def _fin():
        out_ref[...] = out_ref[...] * (1.0 / N)


def _col_mean(node_hidden):
    return pl.pallas_call(
        _mean_body,
        grid=(N // ROW_BLK,),
        in_specs=[pl.BlockSpec((ROW_BLK, D_HID), lambda i: (i, 0))],
        out_specs=pl.BlockSpec((1, D_HID), lambda i: (0, 0)),
        out_shape=jax.ShapeDtypeStruct((1, D_HID), jnp.float32),
    )(node_hidden)


def _bdot(a, b):
    return jnp.dot(a.astype(jnp.bfloat16), b.astype(jnp.bfloat16),
                   preferred_element_type=jnp.float32)


def _mlp_body(last_relu, feat_ref, hid_ref, mean_ref,
              W1, b1, W2, b2, W3, b3, out_ref, *maybe_pack_ref):
    x = jnp.concatenate(
        [feat_ref[...], hid_ref[...],
         jnp.broadcast_to(mean_ref[...], (ROW_BLK, D_HID))], axis=1)
    h = _bdot(x, W1[...]) + b1[...]
    h = jnp.maximum(_bdot(h, W2[...]) + b2[...], 0.0)
    o = _bdot(h, W3[...]) + b3[...]
    o = jnp.maximum(o, 0.0) if last_relu else o
    out_ref[...] = o
    if maybe_pack_ref:
        ob = o.astype(jnp.bfloat16)
        lo = jax.lax.bitcast_convert_type(
            ob[:, :N_OBJ // 2], jnp.uint16).astype(jnp.uint32)
        hi = jax.lax.bitcast_convert_type(
            ob[:, N_OBJ // 2:], jnp.uint16).astype(jnp.uint32)
        maybe_pack_ref[0][...] = ((hi << 16) | lo).astype(jnp.int32)


def _full(shape):
    return pl.BlockSpec(shape, lambda i: tuple(0 for _ in shape))


def _mlp(last_relu, d_out, packed_copy, node_features, node_hidden, mean_h,
         W1, b1, W2, b2, W3, b3):
    out_specs = [pl.BlockSpec((ROW_BLK, d_out), lambda i: (i, 0))]
    out_shape = [jax.ShapeDtypeStruct((N, d_out), jnp.float32)]
    if packed_copy:
        out_specs.append(pl.BlockSpec((ROW_BLK, d_out // 2), lambda i: (i, 0)))
        out_shape.append(jax.ShapeDtypeStruct((N, d_out // 2), jnp.int32))
    return pl.pallas_call(
        functools.partial(_mlp_body, last_relu),
        grid=(N // ROW_BLK,),
        in_specs=[
            pl.BlockSpec((ROW_BLK, D_FEAT), lambda i: (i, 0)),
            pl.BlockSpec((ROW_BLK, D_HID), lambda i: (i, 0)),
            _full((1, D_HID)),
            _full((D_FEAT + 2 * D_HID, H)), _full((1, H)),
            _full((H, H)), _full((1, H)),
            _full((H, d_out)), _full((1, d_out)),
        ],
        out_specs=out_specs,
        out_shape=out_shape,
    )(node_features, node_hidden, mean_h, W1, b1, W2, b2, W3, b3)


def _edge_body(table_hbm, ei_hbm, out_hbm,
               idx_s, idx_d, rs_a, rd_a, rs_b, rd_b, out_v,
               ss_a, sd_a, ss_b, sd_b):
    wid = lax.axis_index("s") * SC_CORES + lax.axis_index("c")
    base = wid * PER_W
    pltpu.sync_copy(ei_hbm.at[0, pl.ds(base, PER_W)], idx_s)
    pltpu.sync_copy(ei_hbm.at[1, pl.ds(base, PER_W)], idx_d)
    lane = lax.iota(jnp.int32, 16)

    def start(c, rs, rd, ss, sd):
        off = c * CHUNK
        pltpu.make_async_copy(
            table_hbm.at[idx_s.at[pl.ds(off, CHUNK)]], rs, ss).start()
        pltpu.make_async_copy(
            table_hbm.at[idx_d.at[pl.ds(off, CHUNK)]], rd, sd).start()

    def wait(rs, rd, ss, sd):
        pltpu.make_async_copy(
            table_hbm.at[idx_s.at[pl.ds(0, CHUNK)]], rs, ss).wait()
        pltpu.make_async_copy(
            table_hbm.at[idx_d.at[pl.ds(0, CHUNK)]], rd, sd).wait()

    def compute(c, rs, rd):
        @plsc.parallel_loop(0, GROUPS, 1, unroll=5)
        def group_body(g):
            row_ids = g * 16 + lane
            accs = [jnp.zeros((16,), jnp.float32) for _ in range(4)]
            for k in range(N_OBJ // 2):
                col = jnp.bitwise_and(lane + k, N_OBJ // 2 - 1)
                sv = plsc.bitcast(
                    plsc.load_gather(rs, [row_ids, col]), jnp.bfloat16)
                dv = plsc.bitcast(
                    plsc.load_gather(rd, [row_ids, col]), jnp.bfloat16)
                plo, phi = plsc.unpack(
                    sv * dv, format=plsc.PackFormat.INTERLEAVED,
                    preferred_element_type=jnp.float32)
                accs[(2 * k) % 4] = accs[(2 * k) % 4] + plo
                accs[(2 * k + 1) % 4] = accs[(2 * k + 1) % 4] + phi
            out_v[pl.ds(c * CHUNK + g * 16, 16)] = (
                (accs[0] + accs[1]) + (accs[2] + accs[3]))

    start(0, rs_a, rd_a, ss_a, sd_a)

    def pair_body(i, carry):
        c0 = 2 * i

        @pl.when(c0 + 1 < NCH)
        def _():
            start(c0 + 1, rs_b, rd_b, ss_b, sd_b)

        wait(rs_a, rd_a, ss_a, sd_a)
        compute(c0, rs_a, rd_a)

        @pl.when(c0 + 2 < NCH)
        def _():
            start(c0 + 2, rs_a, rd_a, ss_a, sd_a)

        @pl.when(c0 + 1 < NCH)
        def _():
            wait(rs_b, rd_b, ss_b, sd_b)
            compute(c0 + 1, rs_b, rd_b)

        return carry

    lax.fori_loop(0, (NCH + 1) // 2, pair_body, 0)
    pltpu.sync_copy(out_v, out_hbm.at[pl.ds(base, PER_W)])


@functools.cache
def _edge_scores():
    return pl.kernel(
        _edge_body,
        out_type=jax.ShapeDtypeStruct((E,), jnp.float32),
        mesh=plsc.VectorSubcoreMesh(
            core_axis_name="c", subcore_axis_name="s",
            num_cores=SC_CORES, num_subcores=SC_SUBCORES),
        compiler_params=pltpu.CompilerParams(
            needs_layout_passes=False, use_tc_tiling_on_sc=False),
        scratch_types=[
            pltpu.VMEM((PER_W,), jnp.int32),
            pltpu.VMEM((PER_W,), jnp.int32),
            pltpu.VMEM((CHUNK, N_OBJ // 2), jnp.int32),
            pltpu.VMEM((CHUNK, N_OBJ // 2), jnp.int32),
            pltpu.VMEM((CHUNK, N_OBJ // 2), jnp.int32),
            pltpu.VMEM((CHUNK, N_OBJ // 2), jnp.int32),
            pltpu.VMEM((PER_W,), jnp.float32),
            pltpu.SemaphoreType.DMA,
            pltpu.SemaphoreType.DMA,
            pltpu.SemaphoreType.DMA,
            pltpu.SemaphoreType.DMA,
        ],
    )


def kernel(node_features, node_hidden, edge_index,
           obj_W1, obj_b1, obj_W2, obj_b2, obj_W3, obj_b3,
           nc_W1, nc_b1, nc_W2, nc_b2, nc_W3, nc_b3):
    mean_h = _col_mean(node_hidden)
    obj_pred, table = _mlp(
        True, N_OBJ, True, node_features, node_hidden, mean_h,
        obj_W1, obj_b1.reshape(1, H), obj_W2, obj_b2.reshape(1, H),
        obj_W3, obj_b3.reshape(1, N_OBJ))
    edge_pred = _edge_scores()(table, edge_index)
    [node_pred] = _mlp(
        False, N_CLS, False, node_features, node_hidden, mean_h,
        nc_W1, nc_b1.reshape(1, H), nc_W2, nc_b2.reshape(1, H),
        nc_W3, nc_b3.reshape(1, N_CLS))
    return obj_pred, edge_pred, node_pred
